# baseline (speedup 1.0000x reference)
import functools

import jax
import jax.numpy as jnp
from jax import lax
from jax.experimental import pallas as pl
from jax.experimental.pallas import tpu as pltpu
from jax.experimental.pallas import tpu_sc as plsc

_NUM_MASHUP = 100000
_NUM_API = 1000
_BATCH = 4096

_L = 16
_NW = 32
_B_PER_W = _BATCH // _NW
_G = 4
_NPAIR = _B_PER_W // (2 * _G)
_FULL = _NUM_API // _L
_TAIL = _NUM_API - _L


def _sigmoid16(x):
    return 1.0 / (1.0 + jnp.exp(-x))


def _sc_body(idx_hbm, table_hbm, out_hbm,
             idx_v, buf_a, buf_b, oslab, gsem_a, gsem_b, osem):
    wid = lax.axis_index("s") * 2 + lax.axis_index("c")
    base = pl.multiple_of(wid * _B_PER_W, 8)
    pltpu.sync_copy(idx_hbm.at[pl.ds(base, _B_PER_W)],
                    idx_v.at[pl.ds(0, _B_PER_W)])

    lane = lax.iota(jnp.int32, _L)

    def fire4(x, lane_off, buf, sem):
        subs = []
        for jj in range(_G):
            u_j = jnp.sum(jnp.where(lane == lane_off + jj, x, 0))
            blk8 = pl.multiple_of(
                lax.shift_left(lax.shift_right_logical(u_j, 3), 3), 8)
            subs.append(lax.bitwise_and(u_j, 7))
            pltpu.async_copy(table_hbm.at[pl.ds(blk8, 8)], buf.at[jj], sem)
        return subs

    def drain4(buf, sem):
        for jj in range(_G):
            pltpu.make_async_copy(
                table_hbm.at[pl.ds(0, 8)], buf.at[jj], sem).wait()

    def compute4(buf, subs, orow0):
        for jj in range(_G):
            s = subs[jj]

            def chunk(c):
                off = c * _L
                oslab[orow0 + jj, pl.ds(off, _L)] = _sigmoid16(
                    buf[jj, s, pl.ds(off, _L)])

            plsc.parallel_loop(0, _FULL, unroll=8)(chunk)
            oslab[orow0 + jj, pl.ds(_TAIL, _L)] = _sigmoid16(
                buf[jj, s, pl.ds(_TAIL, _L)])

    x0 = idx_v[pl.ds(0, _L)]
    subs_a0 = fire4(x0, 0, buf_a, gsem_a)

    def pair(i, subs_a):
        x = idx_v[pl.ds(pl.multiple_of(i * 8, 8), _L)]
        subs_b = fire4(x, _G, buf_b, gsem_b)

        @pl.when(i > 0)
        def _():
            pltpu.make_async_copy(
                oslab, out_hbm.at[pl.ds(0, 2 * _G)], osem).wait()

        drain4(buf_a, gsem_a)
        compute4(buf_a, subs_a, 0)

        xn = idx_v[pl.ds(pl.multiple_of(i * 8 + 8, 8), _L)]
        subs_an = []
        for jj in range(_G):
            u_j = jnp.sum(jnp.where(lane == jj, xn, 0))
            subs_an.append(lax.bitwise_and(u_j, 7))

        @pl.when(i < _NPAIR - 1)
        def _():
            for jj in range(_G):
                u_j = jnp.sum(jnp.where(lane == jj, xn, 0))
                blk8 = pl.multiple_of(
                    lax.shift_left(lax.shift_right_logical(u_j, 3), 3), 8)
                pltpu.async_copy(
                    table_hbm.at[pl.ds(blk8, 8)], buf_a.at[jj], gsem_a)

        drain4(buf_b, gsem_b)
        compute4(buf_b, subs_b, _G)

        off = pl.multiple_of(base + i * 8, 8)
        pltpu.async_copy(oslab, out_hbm.at[pl.ds(off, 2 * _G)], osem)
        return tuple(subs_an)

    subs_final = lax.fori_loop(0, _NPAIR, pair, tuple(subs_a0))
    del subs_final
    pltpu.make_async_copy(oslab, out_hbm.at[pl.ds(0, 2 * _G)], osem).wait()


@jax.jit
def _pop_sc(u, m2a_mat):
    iota = lax.iota(jnp.int32, _BATCH)
    sorted_u, perm = lax.sort_key_val(u, iota)
    rank = jnp.zeros((_BATCH,), jnp.int32).at[perm].set(iota)
    u = jnp.take(sorted_u, rank)
    mesh = plsc.VectorSubcoreMesh(core_axis_name="c", subcore_axis_name="s")
    kfn = functools.partial(
        pl.kernel,
        mesh=mesh,
        compiler_params=pltpu.CompilerParams(needs_layout_passes=False),
        out_type=jax.ShapeDtypeStruct((_BATCH, _NUM_API), jnp.float32),
        scratch_types=[
            pltpu.VMEM((_B_PER_W + _L,), jnp.int32),
            pltpu.VMEM((_G, 8, _NUM_API), jnp.float32),
            pltpu.VMEM((_G, 8, _NUM_API), jnp.float32),
            pltpu.VMEM((2 * _G, _NUM_API), jnp.float32),
            pltpu.SemaphoreType.DMA,
            pltpu.SemaphoreType.DMA,
            pltpu.SemaphoreType.DMA,
        ],
    )(_sc_body)
    return kfn(u, m2a_mat)


def kernel(u, m2a_mat):
    return _pop_sc(u, m2a_mat)


# speedup vs baseline: 1.1495x; 1.1495x over previous
import functools

import jax
import jax.numpy as jnp
from jax import lax
from jax.experimental import pallas as pl
from jax.experimental.pallas import tpu as pltpu
from jax.experimental.pallas import tpu_sc as plsc

_NUM_MASHUP = 100000
_NUM_API = 1000
_BATCH = 4096

_L = 16
_NW = 32
_B_PER_W = _BATCH // _NW
_G = 4
_NPAIR = _B_PER_W // (2 * _G)
_FULL = _NUM_API // _L
_TAIL = _NUM_API - _L
_SPLIT = 496
_LASTB = (_NUM_MASHUP // 128) * 128


def _sigmoid16(x):
    return 1.0 / (1.0 + jnp.exp(-x))


def _s1_body(su_hbm, tt_hbm, m3_hbm, idx_v, buf_a, buf_b, buf_l, oslab):
    wid = lax.axis_index("s") * 2 + lax.axis_index("c")
    base = pl.multiple_of(wid * _B_PER_W, 8)
    lane = lax.iota(jnp.int32, _L)

    def row(p, r_cur):
        @pl.when(lax.bitwise_and(p, 15) == 0)
        def _():
            off = pl.multiple_of(
                base + lax.shift_left(lax.shift_right_logical(p, 4), 4), 8)
            pltpu.sync_copy(su_hbm.at[pl.ds(off, _L)], idx_v)

        x = idx_v[pl.ds(0, _L)]
        u_p = jnp.sum(jnp.where(lane == lax.bitwise_and(p, 15), x, 0))
        r_new = lax.shift_right_logical(u_p, 7)
        l_in = lax.bitwise_and(u_p, 127)

        @pl.when(jnp.logical_and(r_new != r_cur, r_new <= 780))
        def _():
            col = pl.multiple_of(lax.shift_left(r_new, 7), 128)
            pltpu.sync_copy(tt_hbm.at[pl.ds(0, _SPLIT), pl.ds(col, 128)],
                            buf_a)
            pltpu.sync_copy(
                tt_hbm.at[pl.ds(_SPLIT, _NUM_API - _SPLIT), pl.ds(col, 128)],
                buf_b)

        @pl.when(r_new <= 780)
        def _():
            ls = jnp.full((_L,), l_in, jnp.int32)

            def chunk_a(c):
                v = plsc.load_gather(buf_a, [c * _L + lane, ls])
                oslab[lax.shift_right_logical(c, 3),
                      pl.ds(lax.shift_left(lax.bitwise_and(c, 7), 4), _L)] = v

            plsc.parallel_loop(0, _SPLIT // _L, unroll=4)(chunk_a)

            def chunk_b(c):
                v = plsc.load_gather(buf_b, [c * _L - _SPLIT + lane, ls])
                oslab[lax.shift_right_logical(c, 3),
                      pl.ds(lax.shift_left(lax.bitwise_and(c, 7), 4), _L)] = v

            plsc.parallel_loop(_SPLIT // _L, _FULL, unroll=4)(chunk_b)
            v = plsc.load_gather(buf_b, [_TAIL - _SPLIT + lane, ls])
            oslab[7, pl.ds(_TAIL - 7 * 128, _L)] = v

        @pl.when(r_new == 781)
        def _():
            lb = jnp.full((_L,), u_p - _LASTB, jnp.int32)
            l8 = lax.bitwise_and(lane, 7)
            for kp2 in range(62):
                pltpu.sync_copy(
                    tt_hbm.at[pl.ds(kp2 * 16, 8), pl.ds(_LASTB, 32)], buf_l)
                v_lo = plsc.load_gather(buf_l, [l8, lb])
                pltpu.sync_copy(
                    tt_hbm.at[pl.ds(kp2 * 16 + 8, 8), pl.ds(_LASTB, 32)],
                    buf_l)
                v_hi = plsc.load_gather(buf_l, [l8, lb])
                v = jnp.where(lane < 8, v_lo, v_hi)
                oslab[kp2 >> 3, pl.ds((kp2 & 7) * _L, _L)] = v
            pltpu.sync_copy(
                tt_hbm.at[pl.ds(984, 8), pl.ds(_LASTB, 32)], buf_l)
            v_lo = plsc.load_gather(buf_l, [l8, lb])
            pltpu.sync_copy(
                tt_hbm.at[pl.ds(992, 8), pl.ds(_LASTB, 32)], buf_l)
            v_hi = plsc.load_gather(buf_l, [l8, lb])
            v = jnp.where(lane < 8, v_lo, v_hi)
            oslab[7, pl.ds(_TAIL - 7 * 128, _L)] = v

        p8 = pl.multiple_of(lax.shift_left(base + p, 3), 8)
        pltpu.sync_copy(oslab, m3_hbm.at[pl.ds(p8, 8)])
        return r_new

    lax.fori_loop(0, _B_PER_W, row, jnp.int32(-1))


def _s2_body(rank_hbm, m3_hbm, out_hbm,
             idx_v, buf_a, buf_b, oslab, gsem_a, gsem_b, osem):
    wid = lax.axis_index("s") * 2 + lax.axis_index("c")
    base = pl.multiple_of(wid * _B_PER_W, 8)
    pltpu.sync_copy(rank_hbm.at[pl.ds(base, _B_PER_W)],
                    idx_v.at[pl.ds(0, _B_PER_W)])
    lane = lax.iota(jnp.int32, _L)

    def fire4(x, lane_off, buf, sem):
        for jj in range(_G):
            r_j = jnp.sum(jnp.where(lane == lane_off + jj, x, 0))
            blk8 = pl.multiple_of(lax.shift_left(r_j, 3), 8)
            pltpu.async_copy(m3_hbm.at[pl.ds(blk8, 8)], buf.at[jj], sem)

    def drain4(buf, sem):
        for jj in range(_G):
            pltpu.make_async_copy(
                m3_hbm.at[pl.ds(0, 8)], buf.at[jj], gsem_a if buf is buf_a
                else gsem_b).wait()

    def compute4(buf, orow0):
        for jj in range(_G):
            js = jnp.full((_L,), jj, jnp.int32)

            def chunk(c):
                cidx = c * _L + lane
                v = plsc.load_gather(
                    buf, [js, lax.shift_right_logical(cidx, 7),
                          lax.bitwise_and(cidx, 127)])
                oslab[orow0 + jj, pl.ds(c * _L, _L)] = _sigmoid16(v)

            plsc.parallel_loop(0, _FULL, unroll=4)(chunk)
            cidx = _TAIL + lane
            v = plsc.load_gather(
                buf, [js, lax.shift_right_logical(cidx, 7),
                      lax.bitwise_and(cidx, 127)])
            oslab[orow0 + jj, pl.ds(_TAIL, _L)] = _sigmoid16(v)

    x0 = idx_v[pl.ds(0, _L)]
    fire4(x0, 0, buf_a, gsem_a)

    def pair(i, carry):
        x = idx_v[pl.ds(pl.multiple_of(i * 8, 8), _L)]
        fire4(x, _G, buf_b, gsem_b)

        @pl.when(i > 0)
        def _():
            pltpu.make_async_copy(
                oslab, out_hbm.at[pl.ds(0, 2 * _G)], osem).wait()

        drain4(buf_a, gsem_a)
        compute4(buf_a, 0)

        xn = idx_v[pl.ds(pl.multiple_of(i * 8 + 8, 8), _L)]

        @pl.when(i < _NPAIR - 1)
        def _():
            fire4(xn, 0, buf_a, gsem_a)

        drain4(buf_b, gsem_b)
        compute4(buf_b, _G)

        off = pl.multiple_of(base + i * 8, 8)
        pltpu.async_copy(oslab, out_hbm.at[pl.ds(off, 2 * _G)], osem)
        return carry

    lax.fori_loop(0, _NPAIR, pair, 0)
    pltpu.make_async_copy(oslab, out_hbm.at[pl.ds(0, 2 * _G)], osem).wait()


@jax.jit
def _pop_sc(u, m2a_mat):
    table_t = m2a_mat.T
    iota = lax.iota(jnp.int32, _BATCH)
    sorted_u, perm = lax.sort_key_val(u, iota)
    rank = jnp.zeros((_BATCH,), jnp.int32).at[perm].set(iota)

    mesh = plsc.VectorSubcoreMesh(core_axis_name="c", subcore_axis_name="s")
    s1 = functools.partial(
        pl.kernel,
        mesh=mesh,
        compiler_params=pltpu.CompilerParams(needs_layout_passes=False),
        out_type=jax.ShapeDtypeStruct((_BATCH * 8, 128), jnp.float32),
        scratch_types=[
            pltpu.VMEM((_L,), jnp.int32),
            pltpu.VMEM((_SPLIT, 128), jnp.float32),
            pltpu.VMEM((_NUM_API - _SPLIT, 128), jnp.float32),
            pltpu.VMEM((8, 32), jnp.float32),
            pltpu.VMEM((8, 128), jnp.float32),
        ],
    )(_s1_body)
    m3 = s1(sorted_u, table_t)

    s2 = functools.partial(
        pl.kernel,
        mesh=mesh,
        compiler_params=pltpu.CompilerParams(needs_layout_passes=False),
        out_type=jax.ShapeDtypeStruct((_BATCH, _NUM_API), jnp.float32),
        scratch_types=[
            pltpu.VMEM((_B_PER_W + _L,), jnp.int32),
            pltpu.VMEM((_G, 8, 128), jnp.float32),
            pltpu.VMEM((_G, 8, 128), jnp.float32),
            pltpu.VMEM((2 * _G, _NUM_API), jnp.float32),
            pltpu.SemaphoreType.DMA,
            pltpu.SemaphoreType.DMA,
            pltpu.SemaphoreType.DMA,
        ],
    )(_s2_body)
    return s2(rank, m3)


def kernel(u, m2a_mat):
    return _pop_sc(u, m2a_mat)
